# baseline (speedup 1.0000x reference)
import jax
import jax.numpy as jnp
from jax.experimental import pallas as pl


def _matmul_body(w_ref, x_ref, o_ref):
    o_ref[...] = jnp.dot(w_ref[...], x_ref[...],
                         preferred_element_type=jnp.float32)


def kernel(x, W):
    c_in = x.shape[1]
    h, w_dim = x.shape[2], x.shape[3]
    n = h * w_dim
    c_out = W.shape[0]
    x_flat = x.reshape(c_in, n)

    bn = 1024
    assert n % bn == 0
    y = pl.pallas_call(
        _matmul_body,
        grid=(n // bn,),
        in_specs=[
            pl.BlockSpec((c_out, c_in), lambda j: (0, 0)),
            pl.BlockSpec((c_in, bn), lambda j: (0, j)),
        ],
        out_specs=pl.BlockSpec((c_out, bn), lambda j: (0, j)),
        out_shape=jax.ShapeDtypeStruct((c_out, n), jnp.float32),
    )(W, x_flat)
    return y.reshape(1, c_out, h, w_dim)
